# baseline (speedup 1.0000x reference)
import functools

import jax
import jax.numpy as jnp
import numpy as np
from jax.experimental import pallas as pl
from jax.experimental.pallas import tpu as pltpu

NUM_EXPERTS = 16
DIM = 64
EXPERT_DIM = 128
OUTPUT_DIM = 256
HID = NUM_EXPERTS * EXPERT_DIM

HI = jax.lax.Precision.HIGHEST


def _fold_kernel(sr_ref, tr_ref, rev_ref, c_ikj_ref, c_jki_ref,
                 w2_ref, wd1_ref, b2_ref,
                 m_ref, w2m_ref, w2md_ref, b2cat_ref):
    sr = sr_ref[:]
    tr = tr_ref[:]
    rev = rev_ref[:]
    rs = sr / (jnp.sqrt(jnp.sum(sr * sr)) + 1e-8)
    rt = tr / (jnp.sqrt(jnp.sum(tr * tr)) + 1e-8)

    c_ikj = c_ikj_ref[:]
    c_jki = c_jki_ref[:]

    def contract(c, v):
        return jnp.sum(c * v.reshape(1, 1, DIM), axis=2)

    a1 = contract(c_ikj, rs)
    b1m = contract(c_jki, rs * rev)
    a2 = contract(c_ikj, rt * rev)
    b2m = contract(c_jki, rt)
    m = jnp.dot(jnp.dot(jnp.dot(a1, b1m, precision=HI), a2, precision=HI),
                b2m, precision=HI)
    m_ref[:] = m

    w2m = jnp.dot(w2_ref[:], m, precision=HI)
    w2m_ref[:] = w2m
    md = jnp.dot(m, wd1_ref[:], precision=HI)
    w2md_ref[:] = jnp.dot(w2_ref[:], md, precision=HI)

    b2 = b2_ref[:]
    b2cat_ref[:, 0:DIM] = b2
    b2cat_ref[:, DIM:2 * DIM] = jnp.dot(b2, m, precision=HI)
    b2cat_ref[:, 2 * DIM:] = jnp.dot(b2, md, precision=HI)


def _main_kernel(x_ref, wg_ref, w1_ref, b1_ref, w2cat_ref, b2cat_ref,
                 exp_ref, bd1_ref, wd2_ref, bd2_ref,
                 out_ref, route_ref, g_ref, probs_ref):
    x = x_ref[:]
    logits = jnp.dot(x, wg_ref[:])
    lmax = jnp.max(logits, axis=1, keepdims=True)
    ex = jnp.exp(logits - lmax)
    probs = ex / jnp.sum(ex, axis=1, keepdims=True)
    probs_ref[:] = probs

    m1 = jnp.max(probs, axis=1, keepdims=True)
    idx = jax.lax.broadcasted_iota(jnp.int32, probs.shape, 1)
    first = jnp.min(jnp.where(probs == m1, idx, NUM_EXPERTS), axis=1,
                    keepdims=True)
    p_wo = jnp.where(idx == first, -1.0, probs)
    m2 = jnp.max(p_wo, axis=1, keepdims=True)
    w = jnp.where(probs >= m2, probs, 0.0) / (m1 + m2)

    h = jnp.dot(x, w1_ref[:]) + b1_ref[:]
    h = jax.nn.gelu(h)
    w_exp = jnp.dot(w, exp_ref[:])
    big = (jnp.dot(h * w_exp, w2cat_ref[:])
           + jnp.dot(w, b2cat_ref[:]))
    route_ref[:] = big[:, 0:DIM]
    g_ref[:] = big[:, DIM:2 * DIM]

    hd = jax.nn.gelu(big[:, 2 * DIM:] + bd1_ref[:])
    out_ref[:] = jnp.dot(hd, wd2_ref[:]) + bd2_ref[:]


@functools.partial(jax.jit, static_argnames=("interpret",))
def kernel(u_mem, source_rotor, target_rotor, W_g, W1, b1, W2, b2,
           Wd1, bd1, Wd2, bd2, cayley, rev_signs, interpret=False):
    B = u_mem.shape[0]
    BT = 512
    nb = B // BT

    c_ikj = cayley.transpose(0, 2, 1)
    c_jki = cayley.transpose(1, 2, 0)
    w2_flat = W2.reshape(HID, DIM)
    _, w2m, w2md, b2cat = pl.pallas_call(
        _fold_kernel,
        out_shape=[
            jax.ShapeDtypeStruct((DIM, DIM), jnp.float32),
            jax.ShapeDtypeStruct((HID, DIM), jnp.float32),
            jax.ShapeDtypeStruct((HID, EXPERT_DIM), jnp.float32),
            jax.ShapeDtypeStruct((NUM_EXPERTS, OUTPUT_DIM), jnp.float32),
        ],
        interpret=interpret,
    )(source_rotor.reshape(1, DIM), target_rotor.reshape(1, DIM),
      rev_signs.reshape(1, DIM), c_ikj, c_jki, w2_flat, Wd1, b2)

    w2cat = jnp.concatenate([w2_flat, w2m, w2md], axis=1)

    w1_flat = W1.transpose(1, 0, 2).reshape(DIM, HID)
    b1_flat = b1.reshape(1, HID)
    expand = jnp.asarray(np.kron(np.eye(NUM_EXPERTS, dtype=np.float32),
                                 np.ones((1, EXPERT_DIM), np.float32)))

    tok = lambda i: (i, 0)
    rep = lambda i: (0, 0)
    out, route, g, probs = pl.pallas_call(
        _main_kernel,
        grid=(nb,),
        in_specs=[
            pl.BlockSpec((BT, DIM), tok),
            pl.BlockSpec((DIM, NUM_EXPERTS), rep),
            pl.BlockSpec((DIM, HID), rep),
            pl.BlockSpec((1, HID), rep),
            pl.BlockSpec((HID, OUTPUT_DIM), rep),
            pl.BlockSpec((NUM_EXPERTS, OUTPUT_DIM), rep),
            pl.BlockSpec((NUM_EXPERTS, HID), rep),
            pl.BlockSpec((1, EXPERT_DIM), rep),
            pl.BlockSpec((EXPERT_DIM, OUTPUT_DIM), rep),
            pl.BlockSpec((1, OUTPUT_DIM), rep),
        ],
        out_specs=[
            pl.BlockSpec((BT, OUTPUT_DIM), tok),
            pl.BlockSpec((BT, DIM), tok),
            pl.BlockSpec((BT, DIM), tok),
            pl.BlockSpec((BT, NUM_EXPERTS), tok),
        ],
        out_shape=[
            jax.ShapeDtypeStruct((B, OUTPUT_DIM), jnp.float32),
            jax.ShapeDtypeStruct((B, DIM), jnp.float32),
            jax.ShapeDtypeStruct((B, DIM), jnp.float32),
            jax.ShapeDtypeStruct((B, NUM_EXPERTS), jnp.float32),
        ],
        compiler_params=pltpu.CompilerParams(
            dimension_semantics=("parallel",),
        ),
        interpret=interpret,
    )(u_mem, W_g, w1_flat, b1_flat, w2cat, b2cat,
      expand, bd1.reshape(1, EXPERT_DIM), Wd2,
      bd2.reshape(1, OUTPUT_DIM))
    return (out, route, g, probs)


# speedup vs baseline: 1.3357x; 1.3357x over previous
import functools

import jax
import jax.numpy as jnp
import numpy as np
from jax.experimental import pallas as pl
from jax.experimental.pallas import tpu as pltpu

NUM_EXPERTS = 16
DIM = 64
EXPERT_DIM = 128
OUTPUT_DIM = 256
HID = NUM_EXPERTS * EXPERT_DIM

_GELU_C = float(np.sqrt(2.0 / np.pi))
_GELU_A = 0.044715


def _gelu(x):
    dt = x.dtype
    x2 = x * x
    z = x * (dt.type(_GELU_C) + dt.type(_GELU_C * _GELU_A) * x2)
    xh = dt.type(0.5) * x
    return xh * jnp.tanh(z) + xh


def _fold_kernel(sr_ref, tr_ref, rev_ref, c_ikj_ref, c_jki_ref,
                 a1_ref, b1_ref, a2_ref, b2_ref):
    sr = sr_ref[:]
    tr = tr_ref[:]
    rev = rev_ref[:]
    rs = sr / (jnp.sqrt(jnp.sum(sr * sr)) + 1e-8)
    rt = tr / (jnp.sqrt(jnp.sum(tr * tr)) + 1e-8)

    c_ikj = c_ikj_ref[:]
    c_jki = c_jki_ref[:]

    def contract(c, v):
        return jnp.sum(c * v.reshape(1, 1, DIM), axis=2)

    a1_ref[:] = contract(c_ikj, rs)
    b1_ref[:] = contract(c_jki, rs * rev)
    a2_ref[:] = contract(c_ikj, rt * rev)
    b2_ref[:] = contract(c_jki, rt)


def _main_kernel(x_ref, wg_ref, w1_ref, b1_ref, w2_ref, b2_ref,
                 a1_ref, sb1_ref, a2_ref, sb2_ref,
                 wd1_ref, bd1_ref, wd2_ref, bd2_ref,
                 out_ref, route_ref, g_ref, probs_ref):
    x = x_ref[:]
    logits = jnp.dot(x, wg_ref[:])
    lmax = jnp.max(logits, axis=1, keepdims=True)
    ex = jnp.exp(logits - lmax)
    probs = ex / jnp.sum(ex, axis=1, keepdims=True)
    probs_ref[:] = probs

    m1 = jnp.max(probs, axis=1, keepdims=True)
    idx = jax.lax.broadcasted_iota(jnp.int32, probs.shape, 1)
    first = jnp.min(jnp.where(probs == m1, idx, NUM_EXPERTS), axis=1,
                    keepdims=True)
    p_wo = jnp.where(idx == first, -1.0, probs)
    m2 = jnp.max(p_wo, axis=1, keepdims=True)
    w = jnp.where(probs >= m2, probs, 0.0) / (m1 + m2)

    h = jnp.dot(x.astype(jnp.bfloat16), w1_ref[:],
                preferred_element_type=jnp.float32) + b1_ref[:]
    h = _gelu(h.astype(jnp.bfloat16))
    wb = w.astype(jnp.bfloat16)
    hw = jnp.concatenate(
        [h[:, e * EXPERT_DIM:(e + 1) * EXPERT_DIM] * wb[:, e:e + 1]
         for e in range(NUM_EXPERTS)], axis=1)
    route = (jnp.dot(hw, w2_ref[:], preferred_element_type=jnp.float32)
             + jnp.dot(w, b2_ref[:]))
    route_ref[:] = route

    t1 = jnp.dot(route, a1_ref[:])
    inv = jnp.dot(t1, sb1_ref[:])
    t2 = jnp.dot(inv, a2_ref[:])
    g = jnp.dot(t2, sb2_ref[:])
    g_ref[:] = g

    hd = _gelu(jnp.dot(g, wd1_ref[:]) + bd1_ref[:])
    out_ref[:] = jnp.dot(hd, wd2_ref[:]) + bd2_ref[:]


@functools.partial(jax.jit, static_argnames=("interpret",))
def kernel(u_mem, source_rotor, target_rotor, W_g, W1, b1, W2, b2,
           Wd1, bd1, Wd2, bd2, cayley, rev_signs, interpret=False):
    B = u_mem.shape[0]
    BT = 1024
    nb = B // BT

    c_ikj = cayley.transpose(0, 2, 1)
    c_jki = cayley.transpose(1, 2, 0)
    a1, sb1, a2, sb2 = pl.pallas_call(
        _fold_kernel,
        out_shape=[jax.ShapeDtypeStruct((DIM, DIM), jnp.float32)] * 4,
        interpret=interpret,
    )(source_rotor.reshape(1, DIM), target_rotor.reshape(1, DIM),
      rev_signs.reshape(1, DIM), c_ikj, c_jki)

    w1_flat = W1.transpose(1, 0, 2).reshape(DIM, HID).astype(jnp.bfloat16)
    b1_flat = b1.reshape(1, HID)
    w2_flat = W2.reshape(HID, DIM)

    tok = lambda i: (i, 0)
    rep = lambda i: (0, 0)
    out, route, g, probs = pl.pallas_call(
        _main_kernel,
        grid=(nb,),
        in_specs=[
            pl.BlockSpec((BT, DIM), tok),
            pl.BlockSpec((DIM, NUM_EXPERTS), rep),
            pl.BlockSpec((DIM, HID), rep),
            pl.BlockSpec((1, HID), rep),
            pl.BlockSpec((HID, DIM), rep),
            pl.BlockSpec((NUM_EXPERTS, DIM), rep),
            pl.BlockSpec((DIM, DIM), rep),
            pl.BlockSpec((DIM, DIM), rep),
            pl.BlockSpec((DIM, DIM), rep),
            pl.BlockSpec((DIM, DIM), rep),
            pl.BlockSpec((DIM, EXPERT_DIM), rep),
            pl.BlockSpec((1, EXPERT_DIM), rep),
            pl.BlockSpec((EXPERT_DIM, OUTPUT_DIM), rep),
            pl.BlockSpec((1, OUTPUT_DIM), rep),
        ],
        out_specs=[
            pl.BlockSpec((BT, OUTPUT_DIM), tok),
            pl.BlockSpec((BT, DIM), tok),
            pl.BlockSpec((BT, DIM), tok),
            pl.BlockSpec((BT, NUM_EXPERTS), tok),
        ],
        out_shape=[
            jax.ShapeDtypeStruct((B, OUTPUT_DIM), jnp.float32),
            jax.ShapeDtypeStruct((B, DIM), jnp.float32),
            jax.ShapeDtypeStruct((B, DIM), jnp.float32),
            jax.ShapeDtypeStruct((B, NUM_EXPERTS), jnp.float32),
        ],
        compiler_params=pltpu.CompilerParams(
            dimension_semantics=("parallel",),
        ),
        interpret=interpret,
    )(u_mem, W_g, w1_flat, b1_flat, w2_flat, b2, a1, sb1, a2, sb2,
      Wd1, bd1.reshape(1, EXPERT_DIM), Wd2, bd2.reshape(1, OUTPUT_DIM))
    return (out, route, g, probs)


# speedup vs baseline: 1.3627x; 1.0202x over previous
import functools

import jax
import jax.numpy as jnp
import numpy as np
from jax.experimental import pallas as pl
from jax.experimental.pallas import tpu as pltpu

NUM_EXPERTS = 16
DIM = 64
EXPERT_DIM = 128
OUTPUT_DIM = 256
HID = NUM_EXPERTS * EXPERT_DIM

_GELU_C = float(np.sqrt(2.0 / np.pi))
_GELU_A = 0.044715


def _gelu(x):
    dt = x.dtype
    x2 = x * x
    z = x * (dt.type(_GELU_C) + dt.type(_GELU_C * _GELU_A) * x2)
    xh = dt.type(0.5) * x
    return xh * jnp.tanh(z) + xh


def _fold_kernel(sr_ref, tr_ref, rev_ref, c_ikj_ref, c_jki_ref,
                 a1_ref, b1_ref, a2_ref, b2_ref):
    sr = sr_ref[:]
    tr = tr_ref[:]
    rev = rev_ref[:]
    rs = sr / (jnp.sqrt(jnp.sum(sr * sr)) + 1e-8)
    rt = tr / (jnp.sqrt(jnp.sum(tr * tr)) + 1e-8)

    c_ikj = c_ikj_ref[:]
    c_jki = c_jki_ref[:]

    def contract(c, v):
        return jnp.sum(c * v.reshape(1, 1, DIM), axis=2)

    a1_ref[:] = contract(c_ikj, rs)
    b1_ref[:] = contract(c_jki, rs * rev)
    a2_ref[:] = contract(c_ikj, rt * rev)
    b2_ref[:] = contract(c_jki, rt)


def _main_kernel(x_ref, wg_ref, w1_ref, b1_ref, w2_ref, b2_ref,
                 a1_ref, sb1_ref, a2_ref, sb2_ref,
                 wd1_ref, bd1_ref, wd2_ref, bd2_ref,
                 out_ref, route_ref, g_ref, probs_ref):
    x = x_ref[:]
    logits = jnp.dot(x, wg_ref[:])
    lmax = jnp.max(logits, axis=1, keepdims=True)
    ex = jnp.exp(logits - lmax)
    probs = ex / jnp.sum(ex, axis=1, keepdims=True)
    probs_ref[:] = probs

    m1 = jnp.max(probs, axis=1, keepdims=True)
    idx = jax.lax.broadcasted_iota(jnp.int32, probs.shape, 1)
    first = jnp.min(jnp.where(probs == m1, idx, NUM_EXPERTS), axis=1,
                    keepdims=True)
    p_wo = jnp.where(idx == first, -1.0, probs)
    m2 = jnp.max(p_wo, axis=1, keepdims=True)
    w = jnp.where(probs >= m2, probs, 0.0) / (m1 + m2)

    h = jnp.dot(x.astype(jnp.bfloat16), w1_ref[:],
                preferred_element_type=jnp.float32) + b1_ref[:]
    h = _gelu(h.astype(jnp.bfloat16))
    wb = w.astype(jnp.bfloat16)
    hw = jnp.concatenate(
        [h[:, e * EXPERT_DIM:(e + 1) * EXPERT_DIM] * wb[:, e:e + 1]
         for e in range(NUM_EXPERTS)], axis=1)
    route = (jnp.dot(hw, w2_ref[:], preferred_element_type=jnp.float32)
             + jnp.dot(w, b2_ref[:]))
    route_ref[:] = route

    t1 = jnp.dot(route, a1_ref[:])
    inv = jnp.dot(t1, sb1_ref[:])
    t2 = jnp.dot(inv, a2_ref[:])
    g = jnp.dot(t2, sb2_ref[:])
    g_ref[:] = g

    hd = _gelu(jnp.dot(g, wd1_ref[:]) + bd1_ref[:])
    out_ref[:] = jnp.dot(hd, wd2_ref[:]) + bd2_ref[:]


@functools.partial(jax.jit, static_argnames=("interpret",))
def kernel(u_mem, source_rotor, target_rotor, W_g, W1, b1, W2, b2,
           Wd1, bd1, Wd2, bd2, cayley, rev_signs, interpret=False):
    B = u_mem.shape[0]
    BT = 2048
    nb = B // BT

    c_ikj = cayley.transpose(0, 2, 1)
    c_jki = cayley.transpose(1, 2, 0)
    a1, sb1, a2, sb2 = pl.pallas_call(
        _fold_kernel,
        out_shape=[jax.ShapeDtypeStruct((DIM, DIM), jnp.float32)] * 4,
        interpret=interpret,
    )(source_rotor.reshape(1, DIM), target_rotor.reshape(1, DIM),
      rev_signs.reshape(1, DIM), c_ikj, c_jki)

    w1_flat = W1.transpose(1, 0, 2).reshape(DIM, HID).astype(jnp.bfloat16)
    b1_flat = b1.reshape(1, HID)
    w2_flat = W2.reshape(HID, DIM)

    tok = lambda i: (i, 0)
    rep = lambda i: (0, 0)
    out, route, g, probs = pl.pallas_call(
        _main_kernel,
        grid=(nb,),
        in_specs=[
            pl.BlockSpec((BT, DIM), tok),
            pl.BlockSpec((DIM, NUM_EXPERTS), rep),
            pl.BlockSpec((DIM, HID), rep),
            pl.BlockSpec((1, HID), rep),
            pl.BlockSpec((HID, DIM), rep),
            pl.BlockSpec((NUM_EXPERTS, DIM), rep),
            pl.BlockSpec((DIM, DIM), rep),
            pl.BlockSpec((DIM, DIM), rep),
            pl.BlockSpec((DIM, DIM), rep),
            pl.BlockSpec((DIM, DIM), rep),
            pl.BlockSpec((DIM, EXPERT_DIM), rep),
            pl.BlockSpec((1, EXPERT_DIM), rep),
            pl.BlockSpec((EXPERT_DIM, OUTPUT_DIM), rep),
            pl.BlockSpec((1, OUTPUT_DIM), rep),
        ],
        out_specs=[
            pl.BlockSpec((BT, OUTPUT_DIM), tok),
            pl.BlockSpec((BT, DIM), tok),
            pl.BlockSpec((BT, DIM), tok),
            pl.BlockSpec((BT, NUM_EXPERTS), tok),
        ],
        out_shape=[
            jax.ShapeDtypeStruct((B, OUTPUT_DIM), jnp.float32),
            jax.ShapeDtypeStruct((B, DIM), jnp.float32),
            jax.ShapeDtypeStruct((B, DIM), jnp.float32),
            jax.ShapeDtypeStruct((B, NUM_EXPERTS), jnp.float32),
        ],
        compiler_params=pltpu.CompilerParams(
            dimension_semantics=("parallel",),
        ),
        interpret=interpret,
    )(u_mem, W_g, w1_flat, b1_flat, w2_flat, b2, a1, sb1, a2, sb2,
      Wd1, bd1.reshape(1, EXPERT_DIM), Wd2, bd2.reshape(1, OUTPUT_DIM))
    return (out, route, g, probs)


# speedup vs baseline: 1.3656x; 1.0021x over previous
import functools

import jax
import jax.numpy as jnp
import numpy as np
from jax.experimental import pallas as pl
from jax.experimental.pallas import tpu as pltpu

NUM_EXPERTS = 16
DIM = 64
EXPERT_DIM = 128
OUTPUT_DIM = 256
HID = NUM_EXPERTS * EXPERT_DIM

_GELU_C = float(np.sqrt(2.0 / np.pi))
_GELU_A = 0.044715


def _gelu(x):
    dt = x.dtype
    x2 = x * x
    z = x * (dt.type(_GELU_C) + dt.type(_GELU_C * _GELU_A) * x2)
    xh = dt.type(0.5) * x
    return xh * jnp.tanh(z) + xh


def _fold_kernel(sr_ref, tr_ref, rev_ref, c_ikj_ref, c_jki_ref,
                 a1_ref, b1_ref, a2_ref, b2_ref):
    sr = sr_ref[:]
    tr = tr_ref[:]
    rev = rev_ref[:]
    rs = sr / (jnp.sqrt(jnp.sum(sr * sr)) + 1e-8)
    rt = tr / (jnp.sqrt(jnp.sum(tr * tr)) + 1e-8)

    c_ikj = c_ikj_ref[:]
    c_jki = c_jki_ref[:]

    def contract(c, v):
        return jnp.sum(c * v.reshape(1, 1, DIM), axis=2)

    a1_ref[:] = contract(c_ikj, rs)
    b1_ref[:] = contract(c_jki, rs * rev)
    a2_ref[:] = contract(c_ikj, rt * rev)
    b2_ref[:] = contract(c_jki, rt)


def _main_kernel(x_ref, wg_ref, w1_ref, b1_ref, w2_ref, b2_ref,
                 a1_ref, sb1_ref, a2_ref, sb2_ref,
                 wd1_ref, bd1_ref, wd2_ref, bd2_ref,
                 out_ref, route_ref, g_ref, probs_ref):
    x = x_ref[:]
    logits = jnp.dot(x, wg_ref[:])
    lmax = jnp.max(logits, axis=1, keepdims=True)
    ex = jnp.exp(logits - lmax)
    probs = ex / jnp.sum(ex, axis=1, keepdims=True)
    probs_ref[:] = probs

    m1 = jnp.max(probs, axis=1, keepdims=True)
    idx = jax.lax.broadcasted_iota(jnp.int32, probs.shape, 1)
    first = jnp.min(jnp.where(probs == m1, idx, NUM_EXPERTS), axis=1,
                    keepdims=True)
    p_wo = jnp.where(idx == first, -1.0, probs)
    m2 = jnp.max(p_wo, axis=1, keepdims=True)
    w = jnp.where(probs >= m2, probs, 0.0) / (m1 + m2)

    h = jnp.dot(x.astype(jnp.bfloat16), w1_ref[:],
                preferred_element_type=jnp.float32) + b1_ref[:]
    h = _gelu(h.astype(jnp.bfloat16))
    wb = w.astype(jnp.bfloat16)
    hw = jnp.concatenate(
        [h[:, e * EXPERT_DIM:(e + 1) * EXPERT_DIM] * wb[:, e:e + 1]
         for e in range(NUM_EXPERTS)], axis=1)
    route = (jnp.dot(hw, w2_ref[:], preferred_element_type=jnp.float32)
             + jnp.dot(w, b2_ref[:]))
    route_ref[:] = route

    t1 = jnp.dot(route, a1_ref[:])
    inv = jnp.dot(t1, sb1_ref[:])
    t2 = jnp.dot(inv, a2_ref[:])
    g = jnp.dot(t2, sb2_ref[:])
    g_ref[:] = g

    hd = _gelu(jnp.dot(g, wd1_ref[:]) + bd1_ref[:])
    out_ref[:] = jnp.dot(hd, wd2_ref[:]) + bd2_ref[:]


@functools.partial(jax.jit, static_argnames=("interpret",))
def kernel(u_mem, source_rotor, target_rotor, W_g, W1, b1, W2, b2,
           Wd1, bd1, Wd2, bd2, cayley, rev_signs, interpret=False):
    B = u_mem.shape[0]
    BT = 4096
    nb = B // BT

    c_ikj = cayley.transpose(0, 2, 1)
    c_jki = cayley.transpose(1, 2, 0)
    a1, sb1, a2, sb2 = pl.pallas_call(
        _fold_kernel,
        out_shape=[jax.ShapeDtypeStruct((DIM, DIM), jnp.float32)] * 4,
        interpret=interpret,
    )(source_rotor.reshape(1, DIM), target_rotor.reshape(1, DIM),
      rev_signs.reshape(1, DIM), c_ikj, c_jki)

    w1_flat = W1.transpose(1, 0, 2).reshape(DIM, HID).astype(jnp.bfloat16)
    b1_flat = b1.reshape(1, HID)
    w2_flat = W2.reshape(HID, DIM)

    tok = lambda i: (i, 0)
    rep = lambda i: (0, 0)
    out, route, g, probs = pl.pallas_call(
        _main_kernel,
        grid=(nb,),
        in_specs=[
            pl.BlockSpec((BT, DIM), tok),
            pl.BlockSpec((DIM, NUM_EXPERTS), rep),
            pl.BlockSpec((DIM, HID), rep),
            pl.BlockSpec((1, HID), rep),
            pl.BlockSpec((HID, DIM), rep),
            pl.BlockSpec((NUM_EXPERTS, DIM), rep),
            pl.BlockSpec((DIM, DIM), rep),
            pl.BlockSpec((DIM, DIM), rep),
            pl.BlockSpec((DIM, DIM), rep),
            pl.BlockSpec((DIM, DIM), rep),
            pl.BlockSpec((DIM, EXPERT_DIM), rep),
            pl.BlockSpec((1, EXPERT_DIM), rep),
            pl.BlockSpec((EXPERT_DIM, OUTPUT_DIM), rep),
            pl.BlockSpec((1, OUTPUT_DIM), rep),
        ],
        out_specs=[
            pl.BlockSpec((BT, OUTPUT_DIM), tok),
            pl.BlockSpec((BT, DIM), tok),
            pl.BlockSpec((BT, DIM), tok),
            pl.BlockSpec((BT, NUM_EXPERTS), tok),
        ],
        out_shape=[
            jax.ShapeDtypeStruct((B, OUTPUT_DIM), jnp.float32),
            jax.ShapeDtypeStruct((B, DIM), jnp.float32),
            jax.ShapeDtypeStruct((B, DIM), jnp.float32),
            jax.ShapeDtypeStruct((B, NUM_EXPERTS), jnp.float32),
        ],
        compiler_params=pltpu.CompilerParams(
            dimension_semantics=("parallel",),
        ),
        interpret=interpret,
    )(u_mem, W_g, w1_flat, b1_flat, w2_flat, b2, a1, sb1, a2, sb2,
      Wd1, bd1.reshape(1, EXPERT_DIM), Wd2, bd2.reshape(1, OUTPUT_DIM))
    return (out, route, g, probs)
